# Initial kernel scaffold; baseline (speedup 1.0000x reference)
import jax
import jax.numpy as jnp
from jax.experimental import pallas as pl


def kernel(cls, label_cls, pred_bboxes, label_target):
    raise NotImplementedError("write your pallas kernel here")


# speedup vs baseline: 7791.9435x; 7791.9435x over previous
import functools

import jax
import jax.numpy as jnp
from jax.experimental import pallas as pl

N = 3125
NP = 3328
UC = 256
NU = NP // UC
B = 8


def _prep_body(logit_ref, lab_ref, bbox_ref,
               iou_o, prob_o, e1_o, f1_o, e2_o, f2_o, p_o):
    bb = bbox_ref[0]
    x1 = bb[0:1, :]
    y1 = bb[1:2, :]
    x2 = bb[2:3, :]
    y2 = bb[3:4, :]
    tx1 = bb[4:5, :]
    ty1 = bb[5:6, :]
    tx2 = bb[6:7, :]
    ty2 = bb[7:8, :]
    ww = jnp.clip(jnp.minimum(tx2, x2) - jnp.maximum(tx1, x1), 0.0, None)
    hh = jnp.clip(jnp.minimum(ty2, y2) - jnp.maximum(ty1, y1), 0.0, None)
    area = (x2 - x1) * (y2 - y1)
    ta = (tx2 - tx1) * (ty2 - ty1)
    inter = ww * hh
    iou = inter / (area + ta - inter)

    pos = lab_ref[0] > 0.0
    prob = jnp.exp(logit_ref[0])
    pf = jnp.sum(jnp.where(pos, 1.0, 0.0))
    pmin = jnp.min(jnp.where(pos, prob, jnp.inf))
    pmax = jnp.max(jnp.where(pos, prob, -jnp.inf))
    s1 = 0.5 * (pmin + pmax)

    iou_o[0] = iou
    prob_o[0] = prob
    e1_o[0] = jnp.where(pos, jnp.exp(prob - s1), 0.0)
    f1_o[0] = jnp.where(pos, jnp.exp(s1 - prob), 0.0)
    e2_o[0] = jnp.where(pos, jnp.exp(iou - 0.5), 0.0)
    f2_o[0] = jnp.where(pos, jnp.exp(0.5 - iou), 0.0)
    p_o[0] = jnp.broadcast_to(pf, (1, 128))


def _pair_body(k1c_ref, f1c_ref, k1r_ref, e1r_ref,
               k2c_ref, f2c_ref, k2r_ref, e2r_ref,
               s1_o, s2_o):
    u = pl.program_id(1)
    iu = u * UC + jax.lax.broadcasted_iota(jnp.int32, (UC, 1), 0)
    iv = jax.lax.broadcasted_iota(jnp.int32, (1, NP), 1)
    idx_lt = iu < iv

    k1u = k1c_ref[0]
    k1v = k1r_ref[0]
    cond1 = (k1u > k1v) | ((k1u == k1v) & idx_lt)
    t1 = jnp.where(cond1, e1r_ref[0], 0.0)
    part1 = jnp.sum(f1c_ref[0] * jnp.sum(t1, axis=1, keepdims=True))

    k2u = k2c_ref[0]
    k2v = k2r_ref[0]
    cond2 = (k2u > k2v) | ((k2u == k2v) & idx_lt)
    t2 = jnp.where(cond2, e2r_ref[0], 0.0)
    part2 = jnp.sum(f2c_ref[0] * jnp.sum(t2, axis=1, keepdims=True))

    @pl.when(u == 0)
    def _init():
        s1_o[0] = jnp.zeros((1, 128), jnp.float32)
        s2_o[0] = jnp.zeros((1, 128), jnp.float32)

    s1_o[0] += jnp.broadcast_to(part1, (1, 128))
    s2_o[0] += jnp.broadcast_to(part2, (1, 128))


@jax.jit
def kernel(cls, label_cls, pred_bboxes, label_target):
    logit = cls.reshape(B, N, 2)[:, :, 1]
    logit = jnp.pad(logit, ((0, 0), (0, NP - N))).reshape(B, 1, NP)
    lab = jnp.pad(label_cls.reshape(B, N).astype(jnp.float32),
                  ((0, 0), (0, NP - N))).reshape(B, 1, NP)
    tgt = jnp.broadcast_to(label_target[:, :, None], (B, 4, N))
    bbox = jnp.pad(jnp.concatenate([pred_bboxes, tgt], axis=1),
                   ((0, 0), (0, 0), (0, NP - N)))

    row = pl.BlockSpec((1, 1, NP), lambda b: (b, 0, 0))
    iou, prob, e1, f1, e2, f2, pcount = pl.pallas_call(
        _prep_body,
        grid=(B,),
        in_specs=[
            row, row,
            pl.BlockSpec((1, 8, NP), lambda b: (b, 0, 0)),
        ],
        out_specs=[row, row, row, row, row, row,
                   pl.BlockSpec((1, 1, 128), lambda b: (b, 0, 0))],
        out_shape=[jax.ShapeDtypeStruct((B, 1, NP), jnp.float32)] * 6
        + [jax.ShapeDtypeStruct((B, 1, 128), jnp.float32)],
    )(logit, lab, bbox)

    col = pl.BlockSpec((1, UC, 1), lambda b, u: (b, u, 0))
    rowv = pl.BlockSpec((1, 1, NP), lambda b, u: (b, 0, 0))
    acc = pl.BlockSpec((1, 1, 128), lambda b, u: (b, 0, 0))
    sum1, sum2 = pl.pallas_call(
        _pair_body,
        grid=(B, NU),
        in_specs=[col, col, rowv, rowv, col, col, rowv, rowv],
        out_specs=[acc, acc],
        out_shape=[jax.ShapeDtypeStruct((B, 1, 128), jnp.float32)] * 2,
    )(iou.reshape(B, NP, 1), f1.reshape(B, NP, 1),
      iou, e1,
      prob.reshape(B, NP, 1), f2.reshape(B, NP, 1),
      prob, e2)

    p = pcount[:, 0, 0]
    cnt = p * (p - 1.0) * 0.5
    loss1 = sum1[:, 0, 0] / cnt
    loss2 = sum2[:, 0, 0] / cnt
    valid = (p > 1.0) & ~jnp.isnan(loss1) & ~jnp.isnan(loss2)
    l1 = jnp.where(valid, loss1, 0.0)
    l2 = jnp.where(valid, loss2, 0.0)
    nvalid = jnp.sum(valid.astype(jnp.float32))
    final1 = jnp.where(nvalid > 0, jnp.sum(l1) / nvalid, 0.0)
    final2 = jnp.where(nvalid > 0, jnp.sum(l2) / nvalid, 0.0)
    return (final1, final2)
